# baseline (speedup 1.0000x reference)
import functools

import jax
import jax.numpy as jnp
from jax import lax
from jax.experimental import pallas as pl
from jax.experimental.pallas import tpu as pltpu
from jax.experimental.pallas import tpu_sc as plsc

_NC = 2
_NS = 16
_NW = _NC * _NS
_CHUNK = 128


@functools.cache
def _make_gather(batch: int):
    nw = _NS
    bpw = batch // nw
    nchunk = 1
    mesh = plsc.VectorSubcoreMesh(core_axis_name="c", subcore_axis_name="s",
                                  num_cores=1)

    @functools.partial(
        pl.kernel,
        mesh=mesh,
        out_type=jax.ShapeDtypeStruct((batch,), jnp.float32),
        scratch_types=[
            pltpu.VMEM((bpw,), jnp.int32),
            pltpu.VMEM((bpw,), jnp.float32),
            pltpu.SemaphoreType.DMA,
            pltpu.SemaphoreType.DMA,
            pltpu.SemaphoreType.DMA,
        ],
    )
    def gather_kernel(state_hbm, table_hbm, out_hbm, idx_v, rows_v,
                      sem_idx, sem_gat, sem_out):
        wid = lax.axis_index("s")
        base = wid * bpw
        csz = bpw // nchunk
        idx_copies = [
            pltpu.async_copy(state_hbm.at[pl.ds(base + j * csz, csz)],
                             idx_v.at[pl.ds(j * csz, csz)], sem_idx)
            for j in range(nchunk)
        ]
        gathers = []
        for j in range(nchunk):
            idx_copies[j].wait()
            gathers.append(
                pltpu.async_copy(table_hbm.at[idx_v.at[pl.ds(j * csz, csz)]],
                                 rows_v.at[pl.ds(j * csz, csz)], sem_gat))
        out_copies = []
        for j in range(nchunk):
            gathers[j].wait()
            out_copies.append(
                pltpu.async_copy(rows_v.at[pl.ds(j * csz, csz)],
                                 out_hbm.at[pl.ds(base + j * csz, csz)],
                                 sem_out))
        for c in out_copies:
            c.wait()

    return gather_kernel


def kernel(state, rew_matrix):
    state = state.astype(jnp.int32)
    return _make_gather(state.shape[0])(state, rew_matrix)


# speedup vs baseline: 1.0017x; 1.0017x over previous
import functools

import jax
import jax.numpy as jnp
from jax import lax
from jax.experimental import pallas as pl
from jax.experimental.pallas import tpu as pltpu
from jax.experimental.pallas import tpu_sc as plsc

_NC = 2
_NS = 16
_NW = _NC * _NS
_CHUNK = 128


@functools.cache
def _make_gather(batch: int):
    nw = _NS
    bpw = batch // nw
    nchunk = 1
    mesh = plsc.VectorSubcoreMesh(core_axis_name="c", subcore_axis_name="s",
                                  num_cores=1)

    @functools.partial(
        pl.kernel,
        mesh=mesh,
        out_type=jax.ShapeDtypeStruct((batch,), jnp.float32),
        scratch_types=[
            pltpu.VMEM((bpw,), jnp.int32),
            pltpu.VMEM((bpw,), jnp.float32),
            pltpu.SemaphoreType.DMA,
        ],
    )
    def gather_kernel(state_hbm, table_hbm, out_hbm, idx_v, rows_v, sem_gat):
        wid = lax.axis_index("s")
        base = wid * bpw
        pltpu.sync_copy(state_hbm.at[pl.ds(base, bpw)], idx_v)
        pltpu.async_copy(table_hbm.at[idx_v], rows_v, sem_gat).wait()
        pltpu.sync_copy(rows_v, out_hbm.at[pl.ds(base, bpw)])

    return gather_kernel


def kernel(state, rew_matrix):
    state = state.astype(jnp.int32)
    return _make_gather(state.shape[0])(state, rew_matrix)
